# baseline (device time: 35470 ns/iter reference)
import jax
import jax.numpy as jnp
from jax import lax
from jax.experimental import pallas as pl
from jax.experimental.pallas import tpu as pltpu

N_DEV = 16
N_LAYERS = 3
B = 128
D = 128


def kernel(x, Win0, Wout0, Win1, Wout1, Win2, Wout2):
    def body(
        x_ref,
        win0_ref,
        wout0_ref,
        win1_ref,
        wout1_ref,
        win2_ref,
        wout2_ref,
        out_ref,
        comm_ref,
        send_sems,
        recv_sems,
    ):
        my = lax.axis_index("i")

        barrier_sem = pltpu.get_barrier_semaphore()
        for off in range(1, N_DEV):
            pl.semaphore_signal(
                barrier_sem,
                inc=1,
                device_id=((my + off) % N_DEV,),
                device_id_type=pl.DeviceIdType.MESH,
            )
        pl.semaphore_wait(barrier_sem, N_DEV - 1)

        wins = [win0_ref, win1_ref, win2_ref]
        wouts = [wout0_ref, wout1_ref, wout2_ref]

        sends = []
        xv = x_ref[:, :]
        for l in range(N_LAYERS):
            h = jnp.dot(
                xv.astype(jnp.bfloat16),
                wins[l][:, :].astype(jnp.bfloat16),
                preferred_element_type=jnp.float32,
            )
            h = jnp.maximum(h, 0.0)
            partial = jnp.dot(
                h.astype(jnp.bfloat16),
                wouts[l][:, :].astype(jnp.bfloat16),
                preferred_element_type=jnp.float32,
            )

            comm_ref[l, 0, :, :] = partial.astype(jnp.bfloat16)

            for off in range(1, N_DEV):
                rdma = pltpu.make_async_remote_copy(
                    src_ref=comm_ref.at[l, 0],
                    dst_ref=comm_ref.at[l, off],
                    send_sem=send_sems.at[l, off],
                    recv_sem=recv_sems.at[l, off],
                    device_id=((my + off) % N_DEV,),
                    device_id_type=pl.DeviceIdType.MESH,
                )
                rdma.start()
                sends.append(rdma)

            for off in range(1, N_DEV):
                recv = pltpu.make_async_remote_copy(
                    src_ref=comm_ref.at[l, 0],
                    dst_ref=comm_ref.at[l, off],
                    send_sem=send_sems.at[l, off],
                    recv_sem=recv_sems.at[l, off],
                    device_id=(my,),
                    device_id_type=pl.DeviceIdType.MESH,
                )
                recv.wait_recv()

            xv = partial + jnp.sum(
                comm_ref[l, 1:, :, :].astype(jnp.float32), axis=0
            )

        out_ref[:, :] = xv

        for rdma in sends:
            rdma.wait_send()

    return pl.pallas_call(
        body,
        out_shape=jax.ShapeDtypeStruct((B, D), jnp.float32),
        in_specs=[pl.BlockSpec(memory_space=pltpu.VMEM)] * 7,
        out_specs=pl.BlockSpec(memory_space=pltpu.VMEM),
        scratch_shapes=[
            pltpu.VMEM((N_LAYERS, N_DEV, B, D), jnp.bfloat16),
            pltpu.SemaphoreType.DMA((N_LAYERS, N_DEV)),
            pltpu.SemaphoreType.DMA((N_LAYERS, N_DEV)),
        ],
        compiler_params=pltpu.CompilerParams(collective_id=0),
    )(x, Win0, Wout0, Win1, Wout1, Win2, Wout2)


# device time: 32111 ns/iter; 1.1046x vs baseline; 1.1046x over previous
import jax
import jax.numpy as jnp
from jax import lax
from jax.experimental import pallas as pl
from jax.experimental.pallas import tpu as pltpu

N_DEV = 16
N_LAYERS = 3
B = 128
D = 128


def kernel(x, Win0, Wout0, Win1, Wout1, Win2, Wout2):
    def body(
        x_ref,
        win0_ref,
        wout0_ref,
        win1_ref,
        wout1_ref,
        win2_ref,
        wout2_ref,
        out_ref,
        comm_ref,
        send_sems,
        recv_sems,
    ):
        my = lax.axis_index("i")

        barrier_sem = pltpu.get_barrier_semaphore()
        for off in range(1, N_DEV):
            pl.semaphore_signal(
                barrier_sem,
                inc=1,
                device_id=((my + off) % N_DEV,),
                device_id_type=pl.DeviceIdType.MESH,
            )

        wins = [win0_ref, win1_ref, win2_ref]
        wouts = [wout0_ref, wout1_ref, wout2_ref]

        sends = []
        xv = x_ref[:, :]
        for l in range(N_LAYERS):
            h = jnp.dot(
                xv.astype(jnp.bfloat16),
                wins[l][:, :].astype(jnp.bfloat16),
                preferred_element_type=jnp.float32,
            )
            h = jnp.maximum(h, 0.0)
            partial = jnp.dot(
                h.astype(jnp.bfloat16),
                wouts[l][:, :].astype(jnp.bfloat16),
                preferred_element_type=jnp.float32,
            )

            comm_ref[l, 0, :, :] = partial.astype(jnp.bfloat16)

            if l == 0:
                pl.semaphore_wait(barrier_sem, N_DEV - 1)

            for off in range(1, N_DEV):
                rdma = pltpu.make_async_remote_copy(
                    src_ref=comm_ref.at[l, 0],
                    dst_ref=comm_ref.at[l, off],
                    send_sem=send_sems.at[l, off],
                    recv_sem=recv_sems.at[l, off],
                    device_id=((my + off) % N_DEV,),
                    device_id_type=pl.DeviceIdType.MESH,
                )
                rdma.start()
                sends.append(rdma)

            for off in range(1, N_DEV):
                recv = pltpu.make_async_remote_copy(
                    src_ref=comm_ref.at[l, 0],
                    dst_ref=comm_ref.at[l, off],
                    send_sem=send_sems.at[l, off],
                    recv_sem=recv_sems.at[l, off],
                    device_id=(my,),
                    device_id_type=pl.DeviceIdType.MESH,
                )
                recv.wait_recv()

            xv = partial + jnp.sum(
                comm_ref[l, 1:, :, :].astype(jnp.float32), axis=0
            )

        out_ref[:, :] = xv

        for rdma in sends:
            rdma.wait_send()

    return pl.pallas_call(
        body,
        out_shape=jax.ShapeDtypeStruct((B, D), jnp.float32),
        in_specs=[pl.BlockSpec(memory_space=pltpu.VMEM)] * 7,
        out_specs=pl.BlockSpec(memory_space=pltpu.VMEM),
        scratch_shapes=[
            pltpu.VMEM((N_LAYERS, N_DEV, B, D), jnp.bfloat16),
            pltpu.SemaphoreType.DMA((N_LAYERS, N_DEV)),
            pltpu.SemaphoreType.DMA((N_LAYERS, N_DEV)),
        ],
        compiler_params=pltpu.CompilerParams(collective_id=0),
    )(x, Win0, Wout0, Win1, Wout1, Win2, Wout2)


# device time: 14334 ns/iter; 2.4745x vs baseline; 2.2402x over previous
import jax
import jax.numpy as jnp
from jax import lax
from jax.experimental import pallas as pl
from jax.experimental.pallas import tpu as pltpu

N_DEV = 16
N_Z = 4
N_P = 4
N_LAYERS = 3
B = 128
D = 128


def kernel(x, Win0, Wout0, Win1, Wout1, Win2, Wout2):
    def body(
        x_ref,
        win0_ref,
        wout0_ref,
        win1_ref,
        wout1_ref,
        win2_ref,
        wout2_ref,
        out_ref,
        commz_ref,
        commp_ref,
        sendz_sems,
        recvz_sems,
        sendp_sems,
        recvp_sems,
    ):
        my = lax.axis_index("i")
        my_z = my // N_P
        my_s = my % N_P

        barrier_sem = pltpu.get_barrier_semaphore()
        for off in range(1, N_DEV):
            pl.semaphore_signal(
                barrier_sem,
                inc=1,
                device_id=((my + off) % N_DEV,),
                device_id_type=pl.DeviceIdType.MESH,
            )

        wins = [win0_ref, win1_ref, win2_ref]
        wouts = [wout0_ref, wout1_ref, wout2_ref]

        sends = []

        def wait_recvs(comm_ref, send_sems, recv_sems, l):
            for off in range(1, N_Z):
                recv = pltpu.make_async_remote_copy(
                    src_ref=comm_ref.at[l, 0],
                    dst_ref=comm_ref.at[l, off],
                    send_sem=send_sems.at[l, off],
                    recv_sem=recv_sems.at[l, off],
                    device_id=(my,),
                    device_id_type=pl.DeviceIdType.MESH,
                )
                recv.wait_recv()

        xv = x_ref[:, :]
        for l in range(N_LAYERS):
            h = jnp.dot(
                xv.astype(jnp.bfloat16),
                wins[l][:, :].astype(jnp.bfloat16),
                preferred_element_type=jnp.float32,
            )
            h = jnp.maximum(h, 0.0)
            partial = jnp.dot(
                h.astype(jnp.bfloat16),
                wouts[l][:, :].astype(jnp.bfloat16),
                preferred_element_type=jnp.float32,
            )

            commz_ref[l, 0, :, :] = partial.astype(jnp.bfloat16)
            if l == 0:
                pl.semaphore_wait(barrier_sem, N_DEV - 1)
            for dz in range(1, N_Z):
                tgt = N_P * ((my_z + dz) % N_Z) + my_s
                rdma = pltpu.make_async_remote_copy(
                    src_ref=commz_ref.at[l, 0],
                    dst_ref=commz_ref.at[l, dz],
                    send_sem=sendz_sems.at[l, dz],
                    recv_sem=recvz_sems.at[l, dz],
                    device_id=(tgt,),
                    device_id_type=pl.DeviceIdType.MESH,
                )
                rdma.start()
                sends.append(rdma)
            wait_recvs(commz_ref, sendz_sems, recvz_sems, l)
            colsum = partial + jnp.sum(
                commz_ref[l, 1:, :, :].astype(jnp.float32), axis=0
            )

            commp_ref[l, 0, :, :] = colsum.astype(jnp.bfloat16)
            for ds in range(1, N_P):
                tgt = N_P * my_z + (my_s + ds) % N_P
                rdma = pltpu.make_async_remote_copy(
                    src_ref=commp_ref.at[l, 0],
                    dst_ref=commp_ref.at[l, ds],
                    send_sem=sendp_sems.at[l, ds],
                    recv_sem=recvp_sems.at[l, ds],
                    device_id=(tgt,),
                    device_id_type=pl.DeviceIdType.MESH,
                )
                rdma.start()
                sends.append(rdma)
            wait_recvs(commp_ref, sendp_sems, recvp_sems, l)
            xv = colsum + jnp.sum(
                commp_ref[l, 1:, :, :].astype(jnp.float32), axis=0
            )

        out_ref[:, :] = xv

        for rdma in sends:
            rdma.wait_send()

    return pl.pallas_call(
        body,
        out_shape=jax.ShapeDtypeStruct((B, D), jnp.float32),
        in_specs=[pl.BlockSpec(memory_space=pltpu.VMEM)] * 7,
        out_specs=pl.BlockSpec(memory_space=pltpu.VMEM),
        scratch_shapes=[
            pltpu.VMEM((N_LAYERS, N_Z, B, D), jnp.bfloat16),
            pltpu.VMEM((N_LAYERS, N_P, B, D), jnp.bfloat16),
            pltpu.SemaphoreType.DMA((N_LAYERS, N_Z)),
            pltpu.SemaphoreType.DMA((N_LAYERS, N_Z)),
            pltpu.SemaphoreType.DMA((N_LAYERS, N_P)),
            pltpu.SemaphoreType.DMA((N_LAYERS, N_P)),
        ],
        compiler_params=pltpu.CompilerParams(collective_id=0),
    )(x, Win0, Wout0, Win1, Wout1, Win2, Wout2)
